# Initial kernel scaffold; baseline (speedup 1.0000x reference)
import jax
import jax.numpy as jnp
from jax.experimental import pallas as pl


def kernel(x, edge_index, lin1_w, lin1_b, l0g0_ln_w, l0g0_ln_b, l0g0_wl, l0g0_bl, l0g0_wr, l0g1_ln_w, l0g1_ln_b, l0g1_wl, l0g1_bl, l0g1_wr, l1g0_ln_w, l1g0_ln_b, l1g0_wl, l1g0_bl, l1g0_wr, l1g1_ln_w, l1g1_ln_b, l1g1_wl, l1g1_bl, l1g1_wr, norm_w, norm_b, lin2_w, lin2_b):
    raise NotImplementedError("write your pallas kernel here")


# speedup vs baseline: 4.2532x; 4.2532x over previous
import functools

import jax
import jax.numpy as jnp
from jax import lax
from jax.experimental import pallas as pl
from jax.experimental.pallas import tpu as pltpu
from jax.experimental.pallas import tpu_sc as plsc

N = 10000
E = 320000
D = 128
DH = 256
NC = 2
NS = 16
NW = NC * NS
EPT = E // NW
C = 80
NCHUNK = EPT // C
NPAD = 10240
RPS = NPAD // NS
RB = 2000
GRID = N // RB
RW = RPS // C


def _sc_agg_body(table, srcs, dsts, z128,
                 part_a, part_b,
                 acc, sidx, didx, rows, sem):
    c = lax.axis_index("c")
    s = lax.axis_index("s")
    wid = c * NS + s
    row0 = s * RPS
    pltpu.sync_copy(z128, rows)
    for k in range(RW):
        pltpu.sync_copy(rows, acc.at[pl.ds(row0 + k * C, C)])
    plsc.subcore_barrier()

    base = wid * EPT

    def step(i, carry):
        off = base + i * C
        pltpu.sync_copy(srcs.at[pl.ds(off, C)], sidx)
        pltpu.sync_copy(dsts.at[pl.ds(off, C)], didx)
        pltpu.async_copy(table.at[sidx], rows, sem).wait()
        pltpu.sync_copy(rows, acc.at[didx], add=True)
        return carry

    lax.fori_loop(0, NCHUNK, step, 0)
    plsc.subcore_barrier()

    part = [part_a, part_b]
    for ci in range(NC):
        @pl.when(c == ci)
        def _(ci=ci):
            for k in range(RW):
                r0 = row0 + k * C
                pltpu.sync_copy(acc.at[pl.ds(r0, C)], rows)
                pltpu.sync_copy(rows, part[ci].at[pl.ds(r0, C)])


@functools.cache
def _sc_kernel():
    mesh = plsc.VectorSubcoreMesh(core_axis_name="c", subcore_axis_name="s")
    return pl.kernel(
        _sc_agg_body,
        out_type=(
            jax.ShapeDtypeStruct((NPAD, D), jnp.float32),
            jax.ShapeDtypeStruct((NPAD, D), jnp.float32),
        ),
        mesh=mesh,
        scratch_types=[
            pltpu.VMEM_SHARED((NPAD, D), jnp.float32),
            pltpu.VMEM((C,), jnp.int32),
            pltpu.VMEM((C,), jnp.int32),
            pltpu.VMEM((C, D), jnp.float32),
            pltpu.SemaphoreType.DMA,
        ],
        name="sc_gather_segsum",
    )


def _sc_agg(*args):
    return _sc_kernel()(*args)


def _relu_ln(v, w, b):
    mu = jnp.mean(v, axis=-1, keepdims=True)
    var = jnp.mean((v - mu) ** 2, axis=-1, keepdims=True)
    return jax.nn.relu((v - mu) * lax.rsqrt(var + 1e-5) * w + b)


def _dot_t(a, w):
    return lax.dot_general(a, w, (((1,), (1,)), ((), ())),
                           preferred_element_type=jnp.float32)


def _t0_body(x_ref, w1_ref, b1_ref, lnw_ref, lnb_ref, x0_ref, x1_ref, t0_ref):
    h = _dot_t(x_ref[...], w1_ref[...]) + b1_ref[...]
    x0 = h[:, :D]
    x1 = h[:, D:]
    x0_ref[...] = x0
    x1_ref[...] = x1
    t0_ref[...] = _relu_ln(x1, lnw_ref[...], lnb_ref[...])


def _tmid_body(res_ref, pa_ref, pb_ref, ca_ref, cb_ref, tab_ref,
               wl_ref, bl_ref, wr_ref, lnw_ref, lnb_ref, y_ref, tn_ref):
    cnt = ca_ref[:, 0] + cb_ref[:, 0]
    inv = 1.0 / jnp.maximum(cnt, 1.0)
    agg = (pa_ref[...] + pb_ref[...]) * inv[:, None]
    y = (res_ref[...] + _dot_t(agg, wl_ref[...]) + bl_ref[...]
         + _dot_t(tab_ref[...], wr_ref[...]))
    y_ref[...] = y
    tn_ref[...] = _relu_ln(y, lnw_ref[...], lnb_ref[...])


def _tfin_body(res_ref, pa_ref, pb_ref, ca_ref, cb_ref, tab_ref,
               wl_ref, bl_ref, wr_ref, y2_ref, nw_ref, nb_ref,
               w2_ref, b2_ref, out_ref):
    cnt = ca_ref[:, 0] + cb_ref[:, 0]
    inv = 1.0 / jnp.maximum(cnt, 1.0)
    agg = (pa_ref[...] + pb_ref[...]) * inv[:, None]
    y3 = (res_ref[...] + _dot_t(agg, wl_ref[...]) + bl_ref[...]
          + _dot_t(tab_ref[...], wr_ref[...]))
    h = jnp.concatenate([y2_ref[...], y3], axis=1)
    hn = _relu_ln(h, nw_ref[...], nb_ref[...])
    out_ref[...] = _dot_t(hn, w2_ref[...]) + b2_ref[...]


def _rows(shape):
    return pl.BlockSpec((RB,) + shape[1:], lambda i: (i,) + (0,) * (len(shape) - 1))


def _full(shape):
    return pl.BlockSpec(shape, lambda i: (0,) * len(shape))


_f32 = jnp.float32


def _t0_call(x, w1, b1, lnw, lnb):
    return pl.pallas_call(
        _t0_body,
        grid=(GRID,),
        in_specs=[_rows((N, D)), _full((DH, D)), _full((1, DH)),
                  _full((1, D)), _full((1, D))],
        out_specs=(_rows((N, D)), _rows((N, D)), _rows((N, D))),
        out_shape=(jax.ShapeDtypeStruct((N, D), _f32),) * 3,
        name="tc_lin1_ln",
    )(x, w1, b1, lnw, lnb)


def _tmid_call(res, pa, pb, ca, cb, tab, wl, bl, wr, lnw, lnb):
    return pl.pallas_call(
        _tmid_body,
        grid=(GRID,),
        in_specs=[_rows((N, D)), _rows((N, D)), _rows((N, D)),
                  _rows((N, D)), _rows((N, D)), _rows((N, D)),
                  _full((D, D)), _full((1, D)), _full((D, D)),
                  _full((1, D)), _full((1, D))],
        out_specs=(_rows((N, D)), _rows((N, D))),
        out_shape=(jax.ShapeDtypeStruct((N, D), _f32),) * 2,
        name="tc_block_post",
    )(res, pa, pb, ca, cb, tab, wl, bl, wr, lnw, lnb)


def _tfin_call(res, pa, pb, ca, cb, tab, wl, bl, wr, y2, nw, nb, w2, b2):
    return pl.pallas_call(
        _tfin_body,
        grid=(GRID,),
        in_specs=[_rows((N, D)), _rows((N, D)), _rows((N, D)),
                  _rows((N, D)), _rows((N, D)), _rows((N, D)),
                  _full((D, D)), _full((1, D)), _full((D, D)),
                  _rows((N, D)), _full((1, DH)), _full((1, DH)),
                  _full((D, DH)), _full((1, D))],
        out_specs=_rows((N, D)),
        out_shape=jax.ShapeDtypeStruct((N, D), _f32),
        name="tc_final",
    )(res, pa, pb, ca, cb, tab, wl, bl, wr, y2, nw, nb, w2, b2)


def kernel(x, edge_index, lin1_w, lin1_b,
           l0g0_ln_w, l0g0_ln_b, l0g0_wl, l0g0_bl, l0g0_wr,
           l0g1_ln_w, l0g1_ln_b, l0g1_wl, l0g1_bl, l0g1_wr,
           l1g0_ln_w, l1g0_ln_b, l1g0_wl, l1g0_bl, l1g0_wr,
           l1g1_ln_w, l1g1_ln_b, l1g1_wl, l1g1_bl, l1g1_wr,
           norm_w, norm_b, lin2_w, lin2_b):
    src = edge_index[0].astype(jnp.int32)
    dst = edge_index[1].astype(jnp.int32)
    z128 = jnp.zeros((C, D), _f32)
    ones_tab = jnp.ones((N, D), _f32)

    r2 = lambda v: v.reshape(1, -1)

    ca, cb = _sc_agg(ones_tab, src, dst, z128)

    x0, x1, t0 = _t0_call(x, lin1_w, r2(lin1_b), r2(l0g0_ln_w), r2(l0g0_ln_b))

    pa, pb = _sc_agg(t0, src, dst, z128)
    y0, t1 = _tmid_call(x0, pa, pb, ca, cb, t0,
                        l0g0_wl, r2(l0g0_bl), l0g0_wr,
                        r2(l0g1_ln_w), r2(l0g1_ln_b))

    pa, pb = _sc_agg(t1, src, dst, z128)
    y1, t2 = _tmid_call(x1, pa, pb, ca, cb, t1,
                        l0g1_wl, r2(l0g1_bl), l0g1_wr,
                        r2(l1g0_ln_w), r2(l1g0_ln_b))

    pa, pb = _sc_agg(t2, src, dst, z128)
    y2, t3 = _tmid_call(y0, pa, pb, ca, cb, t2,
                        l1g0_wl, r2(l1g0_bl), l1g0_wr,
                        r2(l1g1_ln_w), r2(l1g1_ln_b))

    pa, pb = _sc_agg(t3, src, dst, z128)
    out = _tfin_call(y1, pa, pb, ca, cb, t3,
                     l1g1_wl, r2(l1g1_bl), l1g1_wr, y2,
                     r2(norm_w), r2(norm_b), lin2_w, r2(lin2_b))
    return out
